# baseline (speedup 1.0000x reference)
import jax
import jax.numpy as jnp
from jax.experimental import pallas as pl

N = 50000
E = 800000
B = 256
HID = 128
HEADS = 2
CPH = HID // HEADS
NUM_LAYERS = 2
TARGETS = ('DAT', 'NET', 'SERT')


def _layer_norm(x, g, b, eps=1e-5):
    m = jnp.mean(x, axis=-1, keepdims=True)
    v = jnp.var(x, axis=-1, keepdims=True)
    return (x - m) / jnp.sqrt(v + eps) * g + b


def _gatv2(h, edge_index, eemb, p, i):
    src = edge_index[0]
    dst = edge_index[1]
    xl = (h @ p[f'gat{i}_Wl'] + p[f'gat{i}_bl']).reshape(-1, HEADS, CPH)
    xr = (h @ p[f'gat{i}_Wr'] + p[f'gat{i}_br']).reshape(-1, HEADS, CPH)
    ee = (eemb @ p[f'gat{i}_We']).reshape(-1, HEADS, CPH)
    m = xl[src] + xr[dst] + ee
    m = jax.nn.leaky_relu(m, 0.2)
    alpha = jnp.sum(m * p[f'gat{i}_att'][None], axis=-1)
    amax = jax.ops.segment_max(alpha, dst, num_segments=N)
    amax = jnp.where(jnp.isfinite(amax), amax, 0.0)
    ex = jnp.exp(alpha - amax[dst])
    denom = jax.ops.segment_sum(ex, dst, num_segments=N)
    a = ex / (denom[dst] + 1e-16)
    msg = xl[src] * a[:, :, None]
    out = jax.ops.segment_sum(msg, dst, num_segments=N)
    return out.reshape(-1, HID) + p[f'gat{i}_bias']


def kernel(x, edge_index, edge_attr, batch, params):
    p = params
    h = _layer_norm(x @ p['node_W'] + p['node_b'], p['node_g'], p['node_beta'])
    h = jax.nn.relu(h)
    ee = _layer_norm(edge_attr @ p['edge_W'] + p['edge_b'], p['edge_g'], p['edge_beta'])
    ee = jax.nn.relu(ee)
    for i in range(NUM_LAYERS):
        hn = _gatv2(h, edge_index, ee, p, i)
        hn = _layer_norm(hn, p[f'norm{i}_g'], p[f'norm{i}_b'])
        h = jax.nn.relu(hn) + h
    sums = jax.ops.segment_sum(h, batch, num_segments=B)
    counts = jax.ops.segment_sum(jnp.ones((h.shape[0],), h.dtype), batch, num_segments=B)
    emb = sums / jnp.maximum(counts, 1.0)[:, None]
    emb = jnp.tanh(emb @ p['read_W'] + p['read_b'])
    outs = []
    for t in TARGETS:
        z = jax.nn.relu(_layer_norm(emb @ p[f'{t}_W1'] + p[f'{t}_b1'], p[f'{t}_g1'], p[f'{t}_be1']))
        z = jax.nn.relu(z @ p[f'{t}_W2'] + p[f'{t}_b2'])
        z = jax.nn.sigmoid(z @ p[f'{t}_W3'] + p[f'{t}_b3'])
        outs.append(z[:, 0])
    return jnp.stack(outs, axis=-1)


# speedup vs baseline: 16.4399x; 16.4399x over previous
import functools

import jax
import jax.numpy as jnp
from jax import lax
from jax.experimental import pallas as pl
from jax.experimental.pallas import tpu as pltpu
from jax.experimental.pallas import tpu_sc as plsc

N = 50000
E = 800000
B = 256
NODE_DIM = 86
EDGE_DIM = 18
HID = 128
EHID = 64
CPH = 64
TARGETS = ('DAT', 'NET', 'SERT')

NC = 2
NS = 16
NW = NC * NS

GCH = 128
EPAD = 802816
NCHUNK = EPAD // GCH
GITER = NCHUNK // NW
SITER = NCHUNK // NS
DSTPAD = 1 << 28

NR = 4
NQ = 12544
NPADN = NR * NQ
ACC_ROWS = NQ + 16
TRASH = NQ
WPT = NQ // NS

_f32 = jnp.float32


def _ln(x, g, b):
    m = jnp.mean(x, axis=-1, keepdims=True)
    d = x - m
    v = jnp.mean(d * d, axis=-1, keepdims=True)
    return d * lax.rsqrt(v + 1e-5) * g + b


def _node_enc(x, W, bv, g, be, Wl, bl, Wr, br):
    R = 2000

    def kern(x_r, W_r, b_r, g_r, be_r, Wl_r, bl_r, Wr_r, br_r, h_r, xl_r, xr_r):
        hv = jnp.dot(x_r[...], W_r[...], preferred_element_type=_f32) + b_r[...]
        hv = jnp.maximum(_ln(hv, g_r[...], be_r[...]), 0.0)
        h_r[...] = hv
        xl_r[...] = jnp.dot(hv, Wl_r[...], preferred_element_type=_f32) + bl_r[...]
        xr_r[...] = jnp.dot(hv, Wr_r[...], preferred_element_type=_f32) + br_r[...]

    full = lambda a: pl.BlockSpec(a.shape, lambda i: (0,) * a.ndim)
    return pl.pallas_call(
        kern,
        grid=(N // R,),
        in_specs=[pl.BlockSpec((R, NODE_DIM), lambda i: (i, 0)),
                  full(W), full(bv), full(g), full(be),
                  full(Wl), full(bl), full(Wr), full(br)],
        out_specs=[pl.BlockSpec((R, HID), lambda i: (i, 0))] * 3,
        out_shape=[jax.ShapeDtypeStruct((N, HID), _f32)] * 3,
    )(x, W, bv, g, be, Wl, bl, Wr, br)


def _edge_enc(ea, W, bv, g, be, W0):
    R = 2048

    def kern(ea_r, W_r, b_r, g_r, be_r, W0_r, ee_r, eeW_r):
        ev = jnp.dot(ea_r[...], W_r[...], preferred_element_type=_f32) + b_r[...]
        ev = jnp.maximum(_ln(ev, g_r[...], be_r[...]), 0.0)
        ee_r[...] = ev
        eeW_r[...] = jnp.dot(ev, W0_r[...], preferred_element_type=_f32)

    full = lambda a: pl.BlockSpec(a.shape, lambda i: (0,) * a.ndim)
    return pl.pallas_call(
        kern,
        grid=(EPAD // R,),
        in_specs=[pl.BlockSpec((R, EDGE_DIM), lambda i: (i, 0)),
                  full(W), full(bv), full(g), full(be), full(W0)],
        out_specs=[pl.BlockSpec((R, EHID), lambda i: (i, 0)),
                   pl.BlockSpec((R, HID), lambda i: (i, 0))],
        out_shape=[jax.ShapeDtypeStruct((EPAD, EHID), _f32),
                   jax.ShapeDtypeStruct((EPAD, HID), _f32)],
    )(ea, W, bv, g, be, W0)


def _ee_proj(ee, W1):
    R = 2048

    def kern(ee_r, W_r, o_r):
        o_r[...] = jnp.dot(ee_r[...], W_r[...], preferred_element_type=_f32)

    return pl.pallas_call(
        kern,
        grid=(EPAD // R,),
        in_specs=[pl.BlockSpec((R, EHID), lambda i: (i, 0)),
                  pl.BlockSpec(W1.shape, lambda i: (0, 0))],
        out_specs=pl.BlockSpec((R, HID), lambda i: (i, 0)),
        out_shape=jax.ShapeDtypeStruct((EPAD, HID), _f32),
    )(ee, W1)


def _exmsg(gs, gr, eeW, att):
    R = 2048

    def kern(gs_r, gr_r, ee_r, att_r, om_r, oe_r):
        gsv = gs_r[...]
        m = gsv + gr_r[...] + ee_r[...]
        m = jnp.where(m >= 0.0, m, 0.2 * m)
        aw = m * att_r[...]
        e0 = jnp.exp(jnp.sum(aw[:, :CPH], axis=-1, keepdims=True))
        e1 = jnp.exp(jnp.sum(aw[:, CPH:], axis=-1, keepdims=True))
        om_r[...] = jnp.concatenate([gsv[:, :CPH] * e0, gsv[:, CPH:] * e1],
                                    axis=1)
        oe_r[...] = jnp.concatenate(
            [e0, e1, jnp.zeros((R, HID - 2), _f32)], axis=1)

    return pl.pallas_call(
        kern,
        grid=(EPAD // R,),
        in_specs=[pl.BlockSpec((R, HID), lambda i: (i, 0))] * 3 +
                 [pl.BlockSpec((1, HID), lambda i: (0, 0))],
        out_specs=[pl.BlockSpec((R, HID), lambda i: (i, 0)),
                   pl.BlockSpec((R, HID), lambda i: (i, 0))],
        out_shape=[jax.ShapeDtypeStruct((EPAD, HID), _f32),
                   jax.ShapeDtypeStruct((EPAD, HID), _f32)],
    )(gs, gr, eeW, att)


def _post(om, oe, h, bias, g, bv, proj):
    R = 2000

    def kern(*refs):
        if proj is None:
            om_r, oe_r, h_r, bias_r, g_r, b_r, ho_r = refs
        else:
            om_r, oe_r, h_r, bias_r, g_r, b_r, Wl_r, bl_r, Wr_r, br_r, \
                ho_r, xl_r, xr_r = refs
        m = om_r[...]
        e = oe_r[...]
        ov = jnp.concatenate(
            [m[:, :CPH] / (e[:, 0:1] + 1e-16),
             m[:, CPH:] / (e[:, 1:2] + 1e-16)], axis=1) + bias_r[...]
        hv = jnp.maximum(_ln(ov, g_r[...], b_r[...]), 0.0) + h_r[...]
        ho_r[...] = hv
        if proj is not None:
            xl_r[...] = jnp.dot(hv, Wl_r[...], preferred_element_type=_f32) + bl_r[...]
            xr_r[...] = jnp.dot(hv, Wr_r[...], preferred_element_type=_f32) + br_r[...]

    full = lambda a: pl.BlockSpec(a.shape, lambda i: (0,) * a.ndim)
    in_specs = [pl.BlockSpec((R, HID), lambda i: (i, 0)),
                pl.BlockSpec((R, HID), lambda i: (i, 0)),
                pl.BlockSpec((R, HID), lambda i: (i, 0)),
                full(bias), full(g), full(bv)]
    args = [om, oe, h, bias, g, bv]
    out_specs = [pl.BlockSpec((R, HID), lambda i: (i, 0))]
    out_shape = [jax.ShapeDtypeStruct((N, HID), _f32)]
    if proj is not None:
        Wl, bl, Wr, br = proj
        in_specs += [full(Wl), full(bl), full(Wr), full(br)]
        args += [Wl, bl, Wr, br]
        out_specs *= 3
        out_shape *= 3
    return pl.pallas_call(
        kern,
        grid=(N // R,),
        in_specs=in_specs,
        out_specs=out_specs,
        out_shape=out_shape,
    )(*args)


def _readout(h, batch2, rW, rb, tp):
    R = 2000
    NB = N // R

    def kern(h_r, b_r, rW_r, rb_r, *rest):
        trefs = rest[:24]
        o_r, acc, cnt = rest[24], rest[25], rest[26]
        i = pl.program_id(0)

        @pl.when(i == 0)
        def _():
            acc[...] = jnp.zeros_like(acc)
            cnt[...] = jnp.zeros_like(cnt)

        onehot = (b_r[...] == lax.broadcasted_iota(jnp.int32, (1, B), 1)
                  ).astype(_f32)
        dn = (((0,), (0,)), ((), ()))
        acc[...] += lax.dot_general(onehot, h_r[...], dn,
                                    preferred_element_type=_f32)
        cnt[...] += lax.dot_general(onehot, jnp.ones((R, HID), _f32), dn,
                                    preferred_element_type=_f32)

        @pl.when(i == NB - 1)
        def _():
            emb = acc[...] / jnp.maximum(cnt[...], 1.0)
            emb = jnp.tanh(jnp.dot(emb, rW_r[...],
                                   preferred_element_type=_f32) + rb_r[...])
            outs = []
            for t in range(3):
                W1, b1, g1, be1, W2, b2, W3, b3 = trefs[8 * t:8 * t + 8]
                z = jnp.dot(emb, W1[...], preferred_element_type=_f32) + b1[...]
                z = jnp.maximum(_ln(z, g1[...], be1[...]), 0.0)
                z = jnp.maximum(jnp.dot(z, W2[...],
                                        preferred_element_type=_f32) + b2[...], 0.0)
                z = jax.nn.sigmoid(jnp.dot(z, W3[...],
                                           preferred_element_type=_f32) + b3[...])
                outs.append(z)
            o_r[...] = jnp.concatenate(outs, axis=1)

    full = lambda a: pl.BlockSpec(a.shape, lambda i: (0,) * a.ndim)
    targs = []
    for arrs in tp:
        targs += list(arrs)
    return pl.pallas_call(
        kern,
        grid=(NB,),
        in_specs=[pl.BlockSpec((R, HID), lambda i: (i, 0)),
                  pl.BlockSpec((R, 1), lambda i: (i, 0)),
                  full(rW), full(rb)] + [full(a) for a in targs],
        out_specs=pl.BlockSpec((B, 3), lambda i: (0, 0)),
        out_shape=jax.ShapeDtypeStruct((B, 3), _f32),
        scratch_shapes=[pltpu.VMEM((B, HID), _f32), pltpu.VMEM((B, HID), _f32)],
    )(h, batch2, rW, rb, *targs)


def _gather2(xl, xr, src, dst):
    mesh = plsc.VectorSubcoreMesh(core_axis_name="c", subcore_axis_name="s")

    @functools.partial(
        pl.kernel,
        out_type=[jax.ShapeDtypeStruct((EPAD, HID), _f32)] * 2,
        mesh=mesh,
        scratch_types=[
            pltpu.VMEM((GCH,), jnp.int32),
            pltpu.VMEM((GCH,), jnp.int32),
            pltpu.VMEM((GCH, HID), _f32),
            pltpu.VMEM((GCH, HID), _f32),
            pltpu.SemaphoreType.DMA,
            pltpu.SemaphoreType.DMA,
        ],
    )
    def k(xl_h, xr_h, src_h, dst_h, gs_h, gr_h, sv, dv, ra, rb, sem1, sem2):
        w = lax.axis_index("s") * NC + lax.axis_index("c")

        def body(i, carry):
            base = (w + i * NW) * GCH
            pltpu.sync_copy(src_h.at[pl.ds(base, GCH)], sv)
            pltpu.sync_copy(dst_h.at[pl.ds(base, GCH)], dv)
            cp1 = pltpu.async_copy(xl_h.at[sv], ra, sem1)
            cp2 = pltpu.async_copy(xr_h.at[dv], rb, sem2)
            cp1.wait()
            cp2.wait()
            pltpu.sync_copy(ra, gs_h.at[pl.ds(base, GCH)])
            pltpu.sync_copy(rb, gr_h.at[pl.ds(base, GCH)])
            return carry

        lax.fori_loop(0, GITER, body, 0)

    return k(xl, xr, src, dst)


def _scatter(vals, dst, zz):
    mesh = plsc.VectorSubcoreMesh(core_axis_name="c", subcore_axis_name="s")

    @functools.partial(
        pl.kernel,
        out_type=jax.ShapeDtypeStruct((NPADN, HID), _f32),
        mesh=mesh,
        scratch_types=[
            pltpu.VMEM((GCH,), jnp.int32),
            pltpu.VMEM((GCH,), jnp.int32),
            pltpu.VMEM((GCH, HID), _f32),
            pltpu.VMEM_SHARED((ACC_ROWS, HID), _f32),
        ],
    )
    def k(vals_h, dst_h, zz_h, out_h, dv, dl, rv, acc):
        c = lax.axis_index("c")
        s = lax.axis_index("s")

        for pq in range(NR // 2):
            q = (NR // 2) * c + pq
            lo = q * NQ
            pltpu.sync_copy(zz_h.at[pl.ds(WPT * s, WPT)],
                            acc.at[pl.ds(WPT * s, WPT)])
            pltpu.sync_copy(zz_h.at[pl.ds(NQ, ACC_ROWS - NQ)],
                            acc.at[pl.ds(NQ, ACC_ROWS - NQ)])
            plsc.subcore_barrier()

            def body(i, carry):
                base = (s + i * NS) * GCH
                pltpu.sync_copy(dst_h.at[pl.ds(base, GCH)], dv)
                pltpu.sync_copy(vals_h.at[pl.ds(base, GCH)], rv)
                for j in range(GCH // 16):
                    d = dv[pl.ds(16 * j, 16)]
                    inr = (d >= lo) & (d < lo + NQ)
                    dl[pl.ds(16 * j, 16)] = jnp.where(
                        inr, d - lo, jnp.int32(TRASH))
                pltpu.sync_copy(rv, acc.at[dl], add=True)
                return carry

            lax.fori_loop(0, SITER, body, 0)
            plsc.subcore_barrier()
            pltpu.sync_copy(acc.at[pl.ds(WPT * s, WPT)],
                            out_h.at[pl.ds(lo + WPT * s, WPT)])
            plsc.subcore_barrier()

    return k(vals, dst, zz)


def kernel(x, edge_index, edge_attr, batch, params):
    p = params
    r2 = lambda a: a.reshape(1, -1)

    src_g = jnp.pad(edge_index[0], (0, EPAD - E))
    dst_g = jnp.pad(edge_index[1], (0, EPAD - E))
    dst_s = jnp.pad(edge_index[1], (0, EPAD - E), constant_values=DSTPAD)
    ea_pad = jnp.pad(edge_attr, ((0, EPAD - E), (0, 0)))

    h, xl, xr = _node_enc(
        x, p['node_W'], r2(p['node_b']), r2(p['node_g']), r2(p['node_beta']),
        p['gat0_Wl'], r2(p['gat0_bl']), p['gat0_Wr'], r2(p['gat0_br']))
    ee, eeW = _edge_enc(
        ea_pad, p['edge_W'], r2(p['edge_b']), r2(p['edge_g']),
        r2(p['edge_beta']), p['gat0_We'])

    zz = jnp.zeros((ACC_ROWS, HID), _f32)

    for i in range(2):
        gs, gr = _gather2(xl, xr, src_g, dst_g)
        fum, fue = _exmsg(gs, gr, eeW, r2(p[f'gat{i}_att']))
        om = _scatter(fum, dst_s, zz)
        oe = _scatter(fue, dst_s, zz)
        if i == 0:
            proj = (p['gat1_Wl'], r2(p['gat1_bl']), p['gat1_Wr'], r2(p['gat1_br']))
            h, xl, xr = _post(om, oe, h, r2(p['gat0_bias']), r2(p['norm0_g']),
                              r2(p['norm0_b']), proj)
            eeW = _ee_proj(ee, p['gat1_We'])
        else:
            (h,) = _post(om, oe, h, r2(p['gat1_bias']), r2(p['norm1_g']),
                         r2(p['norm1_b']), None)

    tp = []
    for t in TARGETS:
        tp.append((p[f'{t}_W1'], r2(p[f'{t}_b1']), r2(p[f'{t}_g1']),
                   r2(p[f'{t}_be1']), p[f'{t}_W2'], r2(p[f'{t}_b2']),
                   p[f'{t}_W3'], r2(p[f'{t}_b3'])))
    out = _readout(h, batch.reshape(N, 1), p['read_W'], r2(p['read_b']), tp)
    return out


# speedup vs baseline: 16.5408x; 1.0061x over previous
import functools

import jax
import jax.numpy as jnp
from jax import lax
from jax.experimental import pallas as pl
from jax.experimental.pallas import tpu as pltpu
from jax.experimental.pallas import tpu_sc as plsc

N = 50000
E = 800000
B = 256
NODE_DIM = 86
EDGE_DIM = 18
HID = 128
EHID = 64
CPH = 64
TARGETS = ('DAT', 'NET', 'SERT')

NC = 2
NS = 16
NW = NC * NS

GCH = 128
EPAD = 802816
NCHUNK = EPAD // GCH
GITER = NCHUNK // NW
SITER = NCHUNK // NS
DSTPAD = 1 << 28

NR = 4
NQ = 12544
NPADN = NR * NQ
ACC_ROWS = NQ + 16
TRASH = NQ
WPT = NQ // NS

_f32 = jnp.float32


def _ln(x, g, b):
    m = jnp.mean(x, axis=-1, keepdims=True)
    d = x - m
    v = jnp.mean(d * d, axis=-1, keepdims=True)
    return d * lax.rsqrt(v + 1e-5) * g + b


def _node_enc(x, W, bv, g, be, Wl, bl, Wr, br):
    R = 2000

    def kern(x_r, W_r, b_r, g_r, be_r, Wl_r, bl_r, Wr_r, br_r, h_r, xl_r, xr_r):
        hv = jnp.dot(x_r[...], W_r[...], preferred_element_type=_f32) + b_r[...]
        hv = jnp.maximum(_ln(hv, g_r[...], be_r[...]), 0.0)
        h_r[...] = hv
        xl_r[...] = jnp.dot(hv, Wl_r[...], preferred_element_type=_f32) + bl_r[...]
        xr_r[...] = jnp.dot(hv, Wr_r[...], preferred_element_type=_f32) + br_r[...]

    full = lambda a: pl.BlockSpec(a.shape, lambda i: (0,) * a.ndim)
    return pl.pallas_call(
        kern,
        grid=(N // R,),
        in_specs=[pl.BlockSpec((R, NODE_DIM), lambda i: (i, 0)),
                  full(W), full(bv), full(g), full(be),
                  full(Wl), full(bl), full(Wr), full(br)],
        out_specs=[pl.BlockSpec((R, HID), lambda i: (i, 0))] * 3,
        out_shape=[jax.ShapeDtypeStruct((N, HID), _f32)] * 3,
    )(x, W, bv, g, be, Wl, bl, Wr, br)


def _edge_enc(ea, W, bv, g, be, W0):
    R = 2048

    def kern(ea_r, W_r, b_r, g_r, be_r, W0_r, ee_r, eeW_r):
        ev = jnp.dot(ea_r[...], W_r[...], preferred_element_type=_f32) + b_r[...]
        ev = jnp.maximum(_ln(ev, g_r[...], be_r[...]), 0.0)
        ee_r[...] = ev
        eeW_r[...] = jnp.dot(ev, W0_r[...], preferred_element_type=_f32)

    full = lambda a: pl.BlockSpec(a.shape, lambda i: (0,) * a.ndim)
    return pl.pallas_call(
        kern,
        grid=(EPAD // R,),
        in_specs=[pl.BlockSpec((R, EDGE_DIM), lambda i: (i, 0)),
                  full(W), full(bv), full(g), full(be), full(W0)],
        out_specs=[pl.BlockSpec((R, EHID), lambda i: (i, 0)),
                   pl.BlockSpec((R, HID), lambda i: (i, 0))],
        out_shape=[jax.ShapeDtypeStruct((EPAD, EHID), _f32),
                   jax.ShapeDtypeStruct((EPAD, HID), _f32)],
    )(ea, W, bv, g, be, W0)


def _ee_proj(ee, W1):
    R = 2048

    def kern(ee_r, W_r, o_r):
        o_r[...] = jnp.dot(ee_r[...], W_r[...], preferred_element_type=_f32)

    return pl.pallas_call(
        kern,
        grid=(EPAD // R,),
        in_specs=[pl.BlockSpec((R, EHID), lambda i: (i, 0)),
                  pl.BlockSpec(W1.shape, lambda i: (0, 0))],
        out_specs=pl.BlockSpec((R, HID), lambda i: (i, 0)),
        out_shape=jax.ShapeDtypeStruct((EPAD, HID), _f32),
    )(ee, W1)


def _exmsg(gs, gr, eeW, att):
    R = 2048

    def kern(gs_r, gr_r, ee_r, att_r, om_r, oe_r):
        gsv = gs_r[...]
        m = gsv + gr_r[...] + ee_r[...]
        m = jnp.where(m >= 0.0, m, 0.2 * m)
        aw = m * att_r[...]
        e0 = jnp.exp(jnp.sum(aw[:, :CPH], axis=-1, keepdims=True))
        e1 = jnp.exp(jnp.sum(aw[:, CPH:], axis=-1, keepdims=True))
        om_r[...] = jnp.concatenate([gsv[:, :CPH] * e0, gsv[:, CPH:] * e1],
                                    axis=1)
        oe_r[...] = jnp.concatenate(
            [e0, e1, jnp.zeros((R, HID - 2), _f32)], axis=1)

    return pl.pallas_call(
        kern,
        grid=(EPAD // R,),
        in_specs=[pl.BlockSpec((R, HID), lambda i: (i, 0))] * 3 +
                 [pl.BlockSpec((1, HID), lambda i: (0, 0))],
        out_specs=[pl.BlockSpec((R, HID), lambda i: (i, 0)),
                   pl.BlockSpec((R, HID), lambda i: (i, 0))],
        out_shape=[jax.ShapeDtypeStruct((EPAD, HID), _f32),
                   jax.ShapeDtypeStruct((EPAD, HID), _f32)],
    )(gs, gr, eeW, att)


def _post(om, oe, h, bias, g, bv, proj):
    R = 2000

    def kern(*refs):
        if proj is None:
            om_r, oe_r, h_r, bias_r, g_r, b_r, ho_r = refs
        else:
            om_r, oe_r, h_r, bias_r, g_r, b_r, Wl_r, bl_r, Wr_r, br_r, \
                ho_r, xl_r, xr_r = refs
        m = om_r[...]
        e = oe_r[...]
        ov = jnp.concatenate(
            [m[:, :CPH] / (e[:, 0:1] + 1e-16),
             m[:, CPH:] / (e[:, 1:2] + 1e-16)], axis=1) + bias_r[...]
        hv = jnp.maximum(_ln(ov, g_r[...], b_r[...]), 0.0) + h_r[...]
        ho_r[...] = hv
        if proj is not None:
            xl_r[...] = jnp.dot(hv, Wl_r[...], preferred_element_type=_f32) + bl_r[...]
            xr_r[...] = jnp.dot(hv, Wr_r[...], preferred_element_type=_f32) + br_r[...]

    full = lambda a: pl.BlockSpec(a.shape, lambda i: (0,) * a.ndim)
    in_specs = [pl.BlockSpec((R, HID), lambda i: (i, 0)),
                pl.BlockSpec((R, HID), lambda i: (i, 0)),
                pl.BlockSpec((R, HID), lambda i: (i, 0)),
                full(bias), full(g), full(bv)]
    args = [om, oe, h, bias, g, bv]
    out_specs = [pl.BlockSpec((R, HID), lambda i: (i, 0))]
    out_shape = [jax.ShapeDtypeStruct((N, HID), _f32)]
    if proj is not None:
        Wl, bl, Wr, br = proj
        in_specs += [full(Wl), full(bl), full(Wr), full(br)]
        args += [Wl, bl, Wr, br]
        out_specs *= 3
        out_shape *= 3
    return pl.pallas_call(
        kern,
        grid=(N // R,),
        in_specs=in_specs,
        out_specs=out_specs,
        out_shape=out_shape,
    )(*args)


def _readout(h, batch2, rW, rb, tp):
    R = 2000
    NB = N // R

    def kern(h_r, b_r, rW_r, rb_r, *rest):
        trefs = rest[:24]
        o_r, acc, cnt = rest[24], rest[25], rest[26]
        i = pl.program_id(0)

        @pl.when(i == 0)
        def _():
            acc[...] = jnp.zeros_like(acc)
            cnt[...] = jnp.zeros_like(cnt)

        onehot = (b_r[...] == lax.broadcasted_iota(jnp.int32, (1, B), 1)
                  ).astype(_f32)
        dn = (((0,), (0,)), ((), ()))
        acc[...] += lax.dot_general(onehot, h_r[...], dn,
                                    preferred_element_type=_f32)
        cnt[...] += lax.dot_general(onehot, jnp.ones((R, HID), _f32), dn,
                                    preferred_element_type=_f32)

        @pl.when(i == NB - 1)
        def _():
            emb = acc[...] / jnp.maximum(cnt[...], 1.0)
            emb = jnp.tanh(jnp.dot(emb, rW_r[...],
                                   preferred_element_type=_f32) + rb_r[...])
            outs = []
            for t in range(3):
                W1, b1, g1, be1, W2, b2, W3, b3 = trefs[8 * t:8 * t + 8]
                z = jnp.dot(emb, W1[...], preferred_element_type=_f32) + b1[...]
                z = jnp.maximum(_ln(z, g1[...], be1[...]), 0.0)
                z = jnp.maximum(jnp.dot(z, W2[...],
                                        preferred_element_type=_f32) + b2[...], 0.0)
                z = jax.nn.sigmoid(jnp.dot(z, W3[...],
                                           preferred_element_type=_f32) + b3[...])
                outs.append(z)
            o_r[...] = jnp.concatenate(outs, axis=1)

    full = lambda a: pl.BlockSpec(a.shape, lambda i: (0,) * a.ndim)
    targs = []
    for arrs in tp:
        targs += list(arrs)
    return pl.pallas_call(
        kern,
        grid=(NB,),
        in_specs=[pl.BlockSpec((R, HID), lambda i: (i, 0)),
                  pl.BlockSpec((R, 1), lambda i: (i, 0)),
                  full(rW), full(rb)] + [full(a) for a in targs],
        out_specs=pl.BlockSpec((B, 3), lambda i: (0, 0)),
        out_shape=jax.ShapeDtypeStruct((B, 3), _f32),
        scratch_shapes=[pltpu.VMEM((B, HID), _f32), pltpu.VMEM((B, HID), _f32)],
    )(h, batch2, rW, rb, *targs)


def _gather2(xl, xr, src, dst):
    mesh = plsc.VectorSubcoreMesh(core_axis_name="c", subcore_axis_name="s")

    @functools.partial(
        pl.kernel,
        out_type=[jax.ShapeDtypeStruct((EPAD, HID), _f32)] * 2,
        mesh=mesh,
        scratch_types=(
            [pltpu.VMEM((GCH,), jnp.int32)] * 4 +
            [pltpu.VMEM((GCH, HID), _f32)] * 4 +
            [pltpu.SemaphoreType.DMA] * 6
        ),
    )
    def k(xl_h, xr_h, src_h, dst_h, gs_h, gr_h,
          sva, dva, svb, dvb, raa, rba, rab, rbb,
          sia, sib, sga, sgb, swa, swb):
        w = lax.axis_index("s") * NC + lax.axis_index("c")

        def body(i, carry):
            ba = (w + (2 * i) * NW) * GCH
            bb = (w + (2 * i + 1) * NW) * GCH
            ia1 = pltpu.async_copy(src_h.at[pl.ds(ba, GCH)], sva, sia)
            ia2 = pltpu.async_copy(dst_h.at[pl.ds(ba, GCH)], dva, sia)
            ib1 = pltpu.async_copy(src_h.at[pl.ds(bb, GCH)], svb, sib)
            ib2 = pltpu.async_copy(dst_h.at[pl.ds(bb, GCH)], dvb, sib)
            ia1.wait(); ia2.wait()
            ga1 = pltpu.async_copy(xl_h.at[sva], raa, sga)
            ga2 = pltpu.async_copy(xr_h.at[dva], rba, sga)
            ib1.wait(); ib2.wait()
            gb1 = pltpu.async_copy(xl_h.at[svb], rab, sgb)
            gb2 = pltpu.async_copy(xr_h.at[dvb], rbb, sgb)
            ga1.wait(); ga2.wait()
            wa1 = pltpu.async_copy(raa, gs_h.at[pl.ds(ba, GCH)], swa)
            wa2 = pltpu.async_copy(rba, gr_h.at[pl.ds(ba, GCH)], swa)
            gb1.wait(); gb2.wait()
            wb1 = pltpu.async_copy(rab, gs_h.at[pl.ds(bb, GCH)], swb)
            wb2 = pltpu.async_copy(rbb, gr_h.at[pl.ds(bb, GCH)], swb)
            wa1.wait(); wa2.wait(); wb1.wait(); wb2.wait()
            return carry

        lax.fori_loop(0, GITER // 2, body, 0)

    return k(xl, xr, src, dst)


def _scatter(vals, dst, zz):
    mesh = plsc.VectorSubcoreMesh(core_axis_name="c", subcore_axis_name="s")

    @functools.partial(
        pl.kernel,
        out_type=jax.ShapeDtypeStruct((NPADN, HID), _f32),
        mesh=mesh,
        scratch_types=[
            pltpu.VMEM((GCH,), jnp.int32),
            pltpu.VMEM((GCH,), jnp.int32),
            pltpu.VMEM((GCH, HID), _f32),
            pltpu.VMEM_SHARED((ACC_ROWS, HID), _f32),
        ],
    )
    def k(vals_h, dst_h, zz_h, out_h, dv, dl, rv, acc):
        c = lax.axis_index("c")
        s = lax.axis_index("s")

        for pq in range(NR // 2):
            q = (NR // 2) * c + pq
            lo = q * NQ
            pltpu.sync_copy(zz_h.at[pl.ds(WPT * s, WPT)],
                            acc.at[pl.ds(WPT * s, WPT)])
            pltpu.sync_copy(zz_h.at[pl.ds(NQ, ACC_ROWS - NQ)],
                            acc.at[pl.ds(NQ, ACC_ROWS - NQ)])
            plsc.subcore_barrier()

            def body(i, carry):
                base = (s + i * NS) * GCH
                pltpu.sync_copy(dst_h.at[pl.ds(base, GCH)], dv)
                pltpu.sync_copy(vals_h.at[pl.ds(base, GCH)], rv)
                for j in range(GCH // 16):
                    d = dv[pl.ds(16 * j, 16)]
                    inr = (d >= lo) & (d < lo + NQ)
                    dl[pl.ds(16 * j, 16)] = jnp.where(
                        inr, d - lo, jnp.int32(TRASH))
                pltpu.sync_copy(rv, acc.at[dl], add=True)
                return carry

            lax.fori_loop(0, SITER, body, 0)
            plsc.subcore_barrier()
            pltpu.sync_copy(acc.at[pl.ds(WPT * s, WPT)],
                            out_h.at[pl.ds(lo + WPT * s, WPT)])
            plsc.subcore_barrier()

    return k(vals, dst, zz)


def kernel(x, edge_index, edge_attr, batch, params):
    p = params
    r2 = lambda a: a.reshape(1, -1)

    src_g = jnp.pad(edge_index[0], (0, EPAD - E))
    dst_g = jnp.pad(edge_index[1], (0, EPAD - E))
    dst_s = jnp.pad(edge_index[1], (0, EPAD - E), constant_values=DSTPAD)
    ea_pad = jnp.pad(edge_attr, ((0, EPAD - E), (0, 0)))

    h, xl, xr = _node_enc(
        x, p['node_W'], r2(p['node_b']), r2(p['node_g']), r2(p['node_beta']),
        p['gat0_Wl'], r2(p['gat0_bl']), p['gat0_Wr'], r2(p['gat0_br']))
    ee, eeW = _edge_enc(
        ea_pad, p['edge_W'], r2(p['edge_b']), r2(p['edge_g']),
        r2(p['edge_beta']), p['gat0_We'])

    zz = jnp.zeros((ACC_ROWS, HID), _f32)

    for i in range(2):
        gs, gr = _gather2(xl, xr, src_g, dst_g)
        fum, fue = _exmsg(gs, gr, eeW, r2(p[f'gat{i}_att']))
        om = _scatter(fum, dst_s, zz)
        oe = _scatter(fue, dst_s, zz)
        if i == 0:
            proj = (p['gat1_Wl'], r2(p['gat1_bl']), p['gat1_Wr'], r2(p['gat1_br']))
            h, xl, xr = _post(om, oe, h, r2(p['gat0_bias']), r2(p['norm0_g']),
                              r2(p['norm0_b']), proj)
            eeW = _ee_proj(ee, p['gat1_We'])
        else:
            (h,) = _post(om, oe, h, r2(p['gat1_bias']), r2(p['norm1_g']),
                         r2(p['norm1_b']), None)

    tp = []
    for t in TARGETS:
        tp.append((p[f'{t}_W1'], r2(p[f'{t}_b1']), r2(p[f'{t}_g1']),
                   r2(p[f'{t}_be1']), p[f'{t}_W2'], r2(p[f'{t}_b2']),
                   p[f'{t}_W3'], r2(p[f'{t}_b3'])))
    out = _readout(h, batch.reshape(N, 1), p['read_W'], r2(p['read_b']), tp)
    return out
